# baseline (speedup 1.0000x reference)
import functools

import jax
import jax.numpy as jnp
from jax import lax
from jax.experimental import pallas as pl
from jax.experimental.pallas import tpu as pltpu
from jax.experimental.pallas import tpu_sc as plsc

PH, PW = 7, 7
QK = PH * PW
NW = 32
B_T = 32
NPAD = NW * B_T
CH = 112
NCH = (B_T * QK) // CH
NBUF = 4
SLAG = 2
MAGIC = 12582912.0


def _make_roi_pool(N, H, W, C):
    mesh = plsc.VectorSubcoreMesh(core_axis_name="c", subcore_axis_name="s")
    num_cores = mesh.num_cores

    sl_c = C // 128

    @functools.partial(
        pl.kernel,
        out_type=jax.ShapeDtypeStruct((NPAD * QK, sl_c, 128), jnp.float32),
        mesh=mesh,
        compiler_params=pltpu.CompilerParams(use_tc_tiling_on_sc=True),
        scratch_types=[
            pltpu.VMEM((B_T,), jnp.float32),
            pltpu.VMEM((B_T,), jnp.float32),
            pltpu.VMEM((B_T,), jnp.float32),
            pltpu.VMEM((B_T,), jnp.float32),
            pltpu.VMEM((NCH, CH), jnp.int32),
            pltpu.VMEM((NCH, CH), jnp.int32),
            [pltpu.VMEM((CH, sl_c, 128), jnp.float32) for _ in range(NBUF)],
            [pltpu.SemaphoreType.DMA for _ in range(NBUF)],
            [pltpu.SemaphoreType.DMA for _ in range(NBUF)],
            pltpu.SemaphoreType.DMA,
        ],
    )
    def roi_pool(fm_hbm, x1_hbm, y1_hbm, x2_hbm, y2_hbm, out_hbm,
                 x1v, y1v, x2v, y2v, gidx, sidx, bufs, gsems, osems, rsem):
        wid = lax.axis_index("s") * num_cores + lax.axis_index("c")
        roi_base = wid * B_T

        cps = [
            pltpu.async_copy(x1_hbm.at[pl.ds(roi_base, B_T)], x1v, rsem),
            pltpu.async_copy(y1_hbm.at[pl.ds(roi_base, B_T)], y1v, rsem),
            pltpu.async_copy(x2_hbm.at[pl.ds(roi_base, B_T)], x2v, rsem),
            pltpu.async_copy(y2_hbm.at[pl.ds(roi_base, B_T)], y2v, rsem),
        ]
        for cp in cps:
            cp.wait()

        lanes = lax.iota(jnp.int32, 16)
        magic = jnp.float32(MAGIC)

        def rnd_clip(f, hi):
            r = (f + magic) - magic
            r = jnp.minimum(jnp.maximum(r, jnp.float32(0.0)), jnp.float32(hi))
            return r.astype(jnp.int32)

        gather_d = {}
        store_d = {}

        def start_store(c):
            gather_d[c].wait()
            store_d[c] = pltpu.async_copy(
                bufs[c % NBUF], out_hbm.at[sidx.at[c]], osems[c % NBUF])

        for g in range(B_T // 16):
            sl = pl.ds(g * 16, 16)
            x1 = x1v[sl]
            y1 = y1v[sl]
            x2 = x2v[sl]
            y2 = y2v[sl]
            stepw = (x2 - x1) / jnp.float32(PW)
            steph = (y2 - y1) / jnp.float32(PH)
            n_vec = roi_base + g * 16 + lanes
            n_cl = jnp.minimum(n_vec, N - 1)
            out0 = n_vec * QK
            wcol = [rnd_clip(x1 + jnp.float32(j) * stepw, W - 1)
                    for j in range(PW)]
            for i in range(PH):
                c = g * PH + i
                hrow = rnd_clip(y1 + jnp.float32(i) * steph, H - 1)
                hpart = (n_cl * H + hrow) * W
                for j in range(PW):
                    dst = pl.ds(j * 16, 16)
                    gidx[c, dst] = hpart + wcol[j]
                    sidx[c, dst] = out0 + (i * PW + j)
                if c >= NBUF:
                    store_d[c - NBUF].wait()
                gather_d[c] = pltpu.async_copy(
                    fm_hbm.at[gidx.at[c]], bufs[c % NBUF], gsems[c % NBUF])
                if c >= SLAG:
                    start_store(c - SLAG)
        for c in range(NCH - SLAG, NCH):
            start_store(c)
        for c in range(NCH - NBUF, NCH):
            store_d[c].wait()

    return roi_pool


def kernel(feature_map, rois):
    N, H, W, C = feature_map.shape
    fm3d = feature_map.reshape(N * H * W, C // 128, 128)
    r = rois.astype(jnp.float32)
    pad = NPAD - N
    x1 = jnp.pad(r[:, 0], (0, pad))
    y1 = jnp.pad(r[:, 1], (0, pad))
    x2 = jnp.pad(r[:, 2], (0, pad))
    y2 = jnp.pad(r[:, 3], (0, pad))
    out = _make_roi_pool(N, H, W, C)(fm3d, x1, y1, x2, y2)
    return out.reshape(NPAD, PH, PW, C)[:N]


# speedup vs baseline: 11.1413x; 11.1413x over previous
import functools

import jax
import jax.numpy as jnp
from jax import lax
from jax.experimental import pallas as pl
from jax.experimental.pallas import tpu as pltpu
from jax.experimental.pallas import tpu_sc as plsc

PH, PW = 7, 7
QK = PH * PW
NW = 32
B_T = 32
NPAD = NW * B_T
NBUF = 8
SLAG = 2
MAGIC = 12582912.0


def _make_roi_pool(N, H, W, C):
    mesh = plsc.VectorSubcoreMesh(core_axis_name="c", subcore_axis_name="s")
    num_cores = mesh.num_cores

    @functools.partial(
        pl.kernel,
        out_type=jax.ShapeDtypeStruct((QK * N, C), jnp.float32),
        mesh=mesh,
        scratch_types=[
            pltpu.VMEM((B_T,), jnp.float32),
            pltpu.VMEM((B_T,), jnp.float32),
            pltpu.VMEM((B_T,), jnp.float32),
            pltpu.VMEM((B_T,), jnp.float32),
            pltpu.VMEM((QK, B_T), jnp.int32),
            pltpu.VMEM((QK, B_T), jnp.int32),
            [pltpu.VMEM((B_T, C), jnp.float32) for _ in range(NBUF)],
            [pltpu.SemaphoreType.DMA for _ in range(NBUF)],
            [pltpu.SemaphoreType.DMA for _ in range(NBUF)],
            pltpu.SemaphoreType.DMA,
        ],
    )
    def roi_pool(fm_hbm, x1_hbm, y1_hbm, x2_hbm, y2_hbm, out_hbm,
                 x1v, y1v, x2v, y2v, gidx, sidx, bufs, gsems, osems, rsem):
        wid = lax.axis_index("s") * num_cores + lax.axis_index("c")
        roi_base = wid * B_T

        cps = [
            pltpu.async_copy(x1_hbm.at[pl.ds(roi_base, B_T)], x1v, rsem),
            pltpu.async_copy(y1_hbm.at[pl.ds(roi_base, B_T)], y1v, rsem),
            pltpu.async_copy(x2_hbm.at[pl.ds(roi_base, B_T)], x2v, rsem),
            pltpu.async_copy(y2_hbm.at[pl.ds(roi_base, B_T)], y2v, rsem),
        ]
        for cp in cps:
            cp.wait()

        lanes = lax.iota(jnp.int32, 16)
        magic = jnp.float32(MAGIC)

        def rnd_clip(f, hi):
            r = (f + magic) - magic
            r = jnp.minimum(jnp.maximum(r, jnp.float32(0.0)), jnp.float32(hi))
            return r.astype(jnp.int32)

        groups = []
        for g in range(B_T // 16):
            sl = pl.ds(g * 16, 16)
            x1 = x1v[sl]
            y1 = y1v[sl]
            x2 = x2v[sl]
            y2 = y2v[sl]
            stepw = (x2 - x1) / jnp.float32(PW)
            steph = (y2 - y1) / jnp.float32(PH)
            n_vec = roi_base + g * 16 + lanes
            m_vec = jnp.where(n_vec < N, n_vec, n_vec - (NPAD - N))
            wcol = [rnd_clip(x1 + jnp.float32(j) * stepw, W - 1)
                    for j in range(PW)]
            groups.append((y1, steph, m_vec * (H * W), m_vec, wcol))

        gather_d = {}
        store_d = {}

        def start_store(c):
            gather_d[c].wait()
            store_d[c] = pltpu.async_copy(
                bufs[c % NBUF], out_hbm.at[sidx.at[c]], osems[c % NBUF])

        for i in range(PH):
            for g, (y1, steph, mpart, m_vec, wcol) in enumerate(groups):
                hrow = rnd_clip(y1 + jnp.float32(i) * steph, H - 1)
                hpart = mpart + hrow * W
                dst = pl.ds(g * 16, 16)
                for j in range(PW):
                    q = i * PW + j
                    gidx[q, dst] = hpart + wcol[j]
                    sidx[q, dst] = m_vec + q * N
            for j in range(PW):
                c = i * PW + j
                if c >= NBUF:
                    store_d[c - NBUF].wait()
                gather_d[c] = pltpu.async_copy(
                    fm_hbm.at[gidx.at[c]], bufs[c % NBUF], gsems[c % NBUF])
                if c >= SLAG:
                    start_store(c - SLAG)
        for c in range(QK - SLAG, QK):
            start_store(c)
        for c in range(QK - NBUF, QK):
            store_d[c].wait()

    return roi_pool


def kernel(feature_map, rois):
    N, H, W, C = feature_map.shape
    fm2d = feature_map.reshape(N * H * W, C)
    r = rois.astype(jnp.float32)
    pad = NPAD - N
    x1 = jnp.concatenate([r[:, 0], r[N - pad:, 0]])
    y1 = jnp.concatenate([r[:, 1], r[N - pad:, 1]])
    x2 = jnp.concatenate([r[:, 2], r[N - pad:, 2]])
    y2 = jnp.concatenate([r[:, 3], r[N - pad:, 3]])
    out = _make_roi_pool(N, H, W, C)(fm2d, x1, y1, x2, y2)
    return jnp.transpose(out.reshape(PH, PW, N, C), (2, 0, 1, 3))


# speedup vs baseline: 11.1790x; 1.0034x over previous
import functools

import jax
import jax.numpy as jnp
from jax import lax
from jax.experimental import pallas as pl
from jax.experimental.pallas import tpu as pltpu
from jax.experimental.pallas import tpu_sc as plsc

PH, PW = 7, 7
QK = PH * PW
NW = 32
B_T = 32
NPAD = NW * B_T
QPC = 2
NCH = (QK + QPC - 1) // QPC
NBUF = 6
SLAG = 2
MAGIC = 12582912.0


def _make_roi_pool(N, H, W, C):
    mesh = plsc.VectorSubcoreMesh(core_axis_name="c", subcore_axis_name="s")
    num_cores = mesh.num_cores

    @functools.partial(
        pl.kernel,
        out_type=jax.ShapeDtypeStruct((QK * N, C), jnp.float32),
        mesh=mesh,
        scratch_types=[
            pltpu.VMEM((B_T,), jnp.float32),
            pltpu.VMEM((B_T,), jnp.float32),
            pltpu.VMEM((B_T,), jnp.float32),
            pltpu.VMEM((B_T,), jnp.float32),
            pltpu.VMEM((NCH, QPC * B_T), jnp.int32),
            pltpu.VMEM((NCH, QPC * B_T), jnp.int32),
            [pltpu.VMEM((QPC * B_T, C), jnp.float32) for _ in range(NBUF)],
            [pltpu.SemaphoreType.DMA for _ in range(NBUF)],
            [pltpu.SemaphoreType.DMA for _ in range(NBUF)],
            pltpu.SemaphoreType.DMA,
        ],
    )
    def roi_pool(fm_hbm, x1_hbm, y1_hbm, x2_hbm, y2_hbm, out_hbm,
                 x1v, y1v, x2v, y2v, gidx, sidx, bufs, gsems, osems, rsem):
        wid = lax.axis_index("s") * num_cores + lax.axis_index("c")
        roi_base = wid * B_T

        cps = [
            pltpu.async_copy(x1_hbm.at[pl.ds(roi_base, B_T)], x1v, rsem),
            pltpu.async_copy(y1_hbm.at[pl.ds(roi_base, B_T)], y1v, rsem),
            pltpu.async_copy(x2_hbm.at[pl.ds(roi_base, B_T)], x2v, rsem),
            pltpu.async_copy(y2_hbm.at[pl.ds(roi_base, B_T)], y2v, rsem),
        ]
        for cp in cps:
            cp.wait()

        lanes = lax.iota(jnp.int32, 16)
        magic = jnp.float32(MAGIC)

        def rnd_clip(f, hi):
            r = (f + magic) - magic
            r = jnp.minimum(jnp.maximum(r, jnp.float32(0.0)), jnp.float32(hi))
            return r.astype(jnp.int32)

        groups = []
        for g in range(B_T // 16):
            sl = pl.ds(g * 16, 16)
            x1 = x1v[sl]
            y1 = y1v[sl]
            x2 = x2v[sl]
            y2 = y2v[sl]
            stepw = (x2 - x1) / jnp.float32(PW)
            steph = (y2 - y1) / jnp.float32(PH)
            n_vec = roi_base + g * 16 + lanes
            m_vec = jnp.where(n_vec < N, n_vec, n_vec - (NPAD - N))
            wcol = [rnd_clip(x1 + jnp.float32(j) * stepw, W - 1)
                    for j in range(PW)]
            groups.append((y1, steph, m_vec * (H * W), m_vec, wcol))

        gather_d = {}
        store_d = {}

        def start_store(c):
            gather_d[c].wait()
            store_d[c] = pltpu.async_copy(
                bufs[c % NBUF], out_hbm.at[sidx.at[c]], osems[c % NBUF])

        def fire(c):
            if c >= NBUF:
                store_d[c - NBUF].wait()
            gather_d[c] = pltpu.async_copy(
                fm_hbm.at[gidx.at[c]], bufs[c % NBUF], gsems[c % NBUF])
            if c >= SLAG:
                start_store(c - SLAG)

        next_fire = 0
        for i in range(PH):
            for g, (y1, steph, mpart, m_vec, wcol) in enumerate(groups):
                hrow = rnd_clip(y1 + jnp.float32(i) * steph, H - 1)
                hpart = mpart + hrow * W
                for j in range(PW):
                    q = i * PW + j
                    dst = pl.ds((q % QPC) * B_T + g * 16, 16)
                    gidx[q // QPC, dst] = hpart + wcol[j]
                    sidx[q // QPC, dst] = m_vec + q * N
                    if q == QK - 1:
                        dst2 = pl.ds(((q + 1) % QPC) * B_T + g * 16, 16)
                        gidx[(q + 1) // QPC, dst2] = hpart + wcol[j]
                        sidx[(q + 1) // QPC, dst2] = m_vec + q * N
            last_built = 7 * i + 6 if i < PH - 1 else NCH * QPC - 1
            while (next_fire < NCH
                   and (next_fire + 1) * QPC - 1 <= last_built):
                fire(next_fire)
                next_fire += 1
        for c in range(NCH - SLAG, NCH):
            start_store(c)
        for c in range(NCH - NBUF, NCH):
            store_d[c].wait()

    return roi_pool


def kernel(feature_map, rois):
    N, H, W, C = feature_map.shape
    fm2d = feature_map.reshape(N * H * W, C)
    r = rois.astype(jnp.float32)
    pad = NPAD - N
    x1 = jnp.concatenate([r[:, 0], r[N - pad:, 0]])
    y1 = jnp.concatenate([r[:, 1], r[N - pad:, 1]])
    x2 = jnp.concatenate([r[:, 2], r[N - pad:, 2]])
    y2 = jnp.concatenate([r[:, 3], r[N - pad:, 3]])
    out = _make_roi_pool(N, H, W, C)(fm2d, x1, y1, x2, y2)
    return jnp.transpose(out.reshape(PH, PW, N, C), (2, 0, 1, 3))


# speedup vs baseline: 11.2086x; 1.0027x over previous
import functools

import jax
import jax.numpy as jnp
from jax import lax
from jax.experimental import pallas as pl
from jax.experimental.pallas import tpu as pltpu
from jax.experimental.pallas import tpu_sc as plsc

PH, PW = 7, 7
QK = PH * PW
NW = 32
B_T = 32
NPAD = NW * B_T
QPC = 2
NCH = (QK + QPC - 1) // QPC
NBUF = 6
SLAG = 2
MAGIC = 12582912.0


def _make_roi_pool(N, H, W, C):
    mesh = plsc.VectorSubcoreMesh(core_axis_name="c", subcore_axis_name="s")
    num_cores = mesh.num_cores

    @functools.partial(
        pl.kernel,
        out_type=jax.ShapeDtypeStruct((QK * N, C), jnp.float32),
        mesh=mesh,
        scratch_types=[
            pltpu.VMEM((4, B_T), jnp.float32),
            pltpu.VMEM((NCH, QPC * B_T), jnp.int32),
            pltpu.VMEM((NCH, QPC * B_T), jnp.int32),
            [pltpu.VMEM((QPC * B_T, C), jnp.float32) for _ in range(NBUF)],
            [pltpu.SemaphoreType.DMA for _ in range(NBUF)],
            [pltpu.SemaphoreType.DMA for _ in range(NBUF)],
            pltpu.SemaphoreType.DMA,
        ],
    )
    def roi_pool(fm_hbm, roi4_hbm, out_hbm,
                 roiv, gidx, sidx, bufs, gsems, osems, rsem):
        wid = lax.axis_index("s") * num_cores + lax.axis_index("c")
        roi_base = wid * B_T

        pltpu.async_copy(roi4_hbm.at[wid], roiv, rsem).wait()

        lanes = lax.iota(jnp.int32, 16)
        magic = jnp.float32(MAGIC)

        def rnd_clip(f, hi):
            r = (f + magic) - magic
            r = jnp.minimum(jnp.maximum(r, jnp.float32(0.0)), jnp.float32(hi))
            return r.astype(jnp.int32)

        groups = []
        for g in range(B_T // 16):
            sl = pl.ds(g * 16, 16)
            x1 = roiv[0, sl]
            y1 = roiv[1, sl]
            x2 = roiv[2, sl]
            y2 = roiv[3, sl]
            stepw = (x2 - x1) / jnp.float32(PW)
            steph = (y2 - y1) / jnp.float32(PH)
            n_vec = roi_base + g * 16 + lanes
            m_vec = jnp.where(n_vec < N, n_vec, n_vec - (NPAD - N))
            wcol = [rnd_clip(x1 + jnp.float32(j) * stepw, W - 1)
                    for j in range(PW)]
            groups.append((y1, steph, m_vec * (H * W), m_vec, wcol))

        gather_d = {}
        store_d = {}

        def start_store(c):
            gather_d[c].wait()
            store_d[c] = pltpu.async_copy(
                bufs[c % NBUF], out_hbm.at[sidx.at[c]], osems[c % NBUF])

        def fire(c):
            if c >= NBUF:
                store_d[c - NBUF].wait()
            gather_d[c] = pltpu.async_copy(
                fm_hbm.at[gidx.at[c]], bufs[c % NBUF], gsems[c % NBUF])
            if c >= SLAG:
                start_store(c - SLAG)

        next_fire = 0
        for i in range(PH):
            for g, (y1, steph, mpart, m_vec, wcol) in enumerate(groups):
                hrow = rnd_clip(y1 + jnp.float32(i) * steph, H - 1)
                hpart = mpart + hrow * W
                for j in range(PW):
                    q = i * PW + j
                    dst = pl.ds((q % QPC) * B_T + g * 16, 16)
                    gidx[q // QPC, dst] = hpart + wcol[j]
                    sidx[q // QPC, dst] = m_vec + q * N
                    if q == QK - 1:
                        dst2 = pl.ds(((q + 1) % QPC) * B_T + g * 16, 16)
                        gidx[(q + 1) // QPC, dst2] = hpart + wcol[j]
                        sidx[(q + 1) // QPC, dst2] = m_vec + q * N
            last_built = 7 * i + 6 if i < PH - 1 else NCH * QPC - 1
            while (next_fire < NCH
                   and (next_fire + 1) * QPC - 1 <= last_built):
                fire(next_fire)
                next_fire += 1
        for c in range(NCH - SLAG, NCH):
            start_store(c)
        for c in range(NCH - NBUF, NCH):
            store_d[c].wait()

    return roi_pool


def kernel(feature_map, rois):
    N, H, W, C = feature_map.shape
    fm2d = feature_map.reshape(N * H * W, C)
    r = rois.astype(jnp.float32)
    pad = NPAD - N
    roi4 = jnp.transpose(
        jnp.concatenate([r, r[N - pad:]]).T.reshape(4, NW, B_T), (1, 0, 2))
    out = _make_roi_pool(N, H, W, C)(fm2d, roi4)
    return jnp.transpose(out.reshape(PH, PW, N, C), (2, 0, 1, 3))
